# baseline (speedup 1.0000x reference)
import functools

import jax
import jax.numpy as jnp
from jax import lax
from jax.experimental import pallas as pl
from jax.experimental.pallas import tpu as pltpu
from jax.experimental.pallas import tpu_sc as plsc


def _oem_merge(lo, hi, r):
    step = r * 2
    if step < hi - lo:
        yield from _oem_merge(lo, hi, step)
        yield from _oem_merge(lo + r, hi, step)
        for i in range(lo + r, hi - r, step):
            yield (i, i + r)
    else:
        yield (lo, lo + r)


def _oem_sort(lo, hi):
    if (hi - lo) >= 1:
        mid = lo + ((hi - lo) // 2)
        yield from _oem_sort(lo, mid)
        yield from _oem_sort(mid + 1, hi)
        yield from _oem_merge(lo, hi, 1)


_SORT32 = tuple(_oem_sort(0, 31))


def _sortnet32(vals):
    vals = list(vals)
    for i, j in _SORT32:
        a, b = vals[i], vals[j]
        vals[i] = jnp.minimum(a, b)
        vals[j] = jnp.maximum(a, b)
    return vals


def _median65(vals):
    A = _sortnet32(vals[0:32])
    Bv = _sortnet32(vals[32:64])
    e = vals[64]
    u31 = jnp.minimum(Bv[31], A[31])
    for i in range(1, 32):
        u31 = jnp.minimum(u31, jnp.maximum(A[i - 1], Bv[31 - i]))
    u32 = jnp.maximum(A[0], Bv[31])
    for i in range(2, 33):
        u32 = jnp.minimum(u32, jnp.maximum(A[i - 1], Bv[32 - i]))
    return jnp.minimum(jnp.maximum(u31, e), u32)


_MM_BLK = 256


def _matmul_body(s_ref, x_ref, o_ref):
    o_ref[...] = lax.dot_general(
        s_ref[...], x_ref[...], (((0,), (1,)), ((), ())),
        preferred_element_type=jnp.float32,
    )


def _matmul_call(S0, x2):
    BF, N_ = x2.shape
    grid = (N_ // _MM_BLK,)
    return pl.pallas_call(
        _matmul_body,
        grid=grid,
        in_specs=[
            pl.BlockSpec((N_, _MM_BLK), lambda i: (0, i)),
            pl.BlockSpec((BF, N_), lambda i: (0, 0)),
        ],
        out_specs=pl.BlockSpec((_MM_BLK, BF), lambda i: (i, 0)),
        out_shape=jax.ShapeDtypeStruct((N_, BF), jnp.float32),
    )(S0, x2)


_NC, _NS = 2, 16
_NW = _NC * _NS
_CHUNK = 104


def _make_gather(rows_total, row_len):
    per_w = rows_total // _NW
    npair = per_w // _CHUNK // 2
    mesh = plsc.VectorSubcoreMesh(core_axis_name="c", subcore_axis_name="s")

    @functools.partial(
        pl.kernel,
        out_type=jax.ShapeDtypeStruct((rows_total, row_len), jnp.float32),
        mesh=mesh,
        scratch_types=[
            pltpu.VMEM((per_w,), jnp.int32),
            pltpu.VMEM((_CHUNK, row_len), jnp.float32),
            pltpu.VMEM((_CHUNK, row_len), jnp.float32),
            pltpu.SemaphoreType.DMA,
            pltpu.SemaphoreType.DMA,
            pltpu.SemaphoreType.DMA,
            pltpu.SemaphoreType.DMA,
        ],
    )
    def gather_k(table_hbm, idx_hbm, out_hbm, idx_all, rows0, rows1,
                 sg0, sg1, ss0, ss1):
        wid = lax.axis_index("s") * _NC + lax.axis_index("c")
        base = pl.multiple_of(wid * per_w, 8)
        pltpu.sync_copy(idx_hbm.at[pl.ds(base, per_w)], idx_all)

        def g_start(c, buf, sem):
            return pltpu.async_copy(
                table_hbm.at[idx_all.at[pl.ds(c * _CHUNK, _CHUNK)]], buf, sem)

        def g_wait(c, buf, sem):
            pltpu.make_async_copy(
                table_hbm.at[idx_all.at[pl.ds(c * _CHUNK, _CHUNK)]], buf, sem
            ).wait()

        def s_start(c, buf, sem):
            off = pl.multiple_of(base + c * _CHUNK, 8)
            return pltpu.async_copy(buf, out_hbm.at[pl.ds(off, _CHUNK)], sem)

        def s_wait(c, buf, sem):
            off = pl.multiple_of(base + c * _CHUNK, 8)
            pltpu.make_async_copy(buf, out_hbm.at[pl.ds(off, _CHUNK)], sem).wait()

        g_start(0, rows0, sg0)
        g_start(1, rows1, sg1)

        def body(i, carry):
            c0 = i * 2
            c1 = c0 + 1
            g_wait(c0, rows0, sg0)
            s_start(c0, rows0, ss0)
            g_wait(c1, rows1, sg1)
            s_start(c1, rows1, ss1)

            @pl.when(i < npair - 1)
            def _():
                s_wait(c0, rows0, ss0)
                g_start(c0 + 2, rows0, sg0)
                s_wait(c1, rows1, ss1)
                g_start(c1 + 2, rows1, sg1)

            return carry

        lax.fori_loop(0, npair, body, 0)
        s_wait(2 * npair - 2, rows0, ss0)
        s_wait(2 * npair - 1, rows1, ss1)

    return gather_k


_MED_SUB = 8
_MED_OUT = 128
_MED_INNER = _MED_OUT // _MED_SUB


def _median_body(w_ref, g_ref, x_ref, o_ref, acc_ref):
    j = pl.program_id(1)
    vals = [g_ref[0, d] for d in range(65)]
    med = _median65(vals)
    acc_ref[pl.ds(j * _MED_SUB, _MED_SUB), :] = med

    @pl.when(j == _MED_INNER - 1)
    def _():
        w0 = w_ref[0, 0]
        w1 = w_ref[0, 1]
        o_ref[...] = w0 * jnp.maximum(x_ref[...], 0.0) + w1 * acc_ref[...].T


def _median_call(weight, G, x2):
    NB, Kp1, _, BF = G.shape
    N_ = NB * _MED_SUB
    grid = (N_ // _MED_OUT, _MED_INNER)
    return pl.pallas_call(
        _median_body,
        grid=grid,
        in_specs=[
            pl.BlockSpec((1, 2), lambda i, j: (0, 0)),
            pl.BlockSpec((1, Kp1, _MED_SUB, BF),
                         lambda i, j: (i * _MED_INNER + j, 0, 0, 0)),
            pl.BlockSpec((BF, _MED_OUT), lambda i, j: (0, i)),
        ],
        out_specs=pl.BlockSpec((BF, _MED_OUT), lambda i, j: (0, i)),
        out_shape=jax.ShapeDtypeStruct((BF, N_), jnp.float32),
        scratch_shapes=[pltpu.VMEM((_MED_OUT, BF), jnp.float32)],
    )(weight, G, x2)


def kernel(x, S, weight, neigh_idx):
    B_, F_, N_ = x.shape
    BF = B_ * F_
    Dp1 = neigh_idx.shape[1]

    x2 = x.reshape(BF, N_)
    cur_T = _matmul_call(S[0], x2)
    idx_flat = neigh_idx.reshape(N_ // _MED_SUB, _MED_SUB, Dp1)
    idx_flat = idx_flat.transpose(0, 2, 1).reshape(-1)
    gather_k = _make_gather(Dp1 * N_, BF)
    G = gather_k(cur_T, idx_flat).reshape(N_ // _MED_SUB, Dp1, _MED_SUB, BF)
    out2 = _median_call(weight, G, x2)
    return out2.reshape(B_, F_, N_)


# speedup vs baseline: 1.5419x; 1.5419x over previous
import functools

import jax
import jax.numpy as jnp
from jax import lax
from jax.experimental import pallas as pl
from jax.experimental.pallas import tpu as pltpu
from jax.experimental.pallas import tpu_sc as plsc


def _oem_merge(lo, hi, r):
    step = r * 2
    if step < hi - lo:
        yield from _oem_merge(lo, hi, step)
        yield from _oem_merge(lo + r, hi, step)
        for i in range(lo + r, hi - r, step):
            yield (i, i + r)
    else:
        yield (lo, lo + r)


def _oem_sort(lo, hi):
    if (hi - lo) >= 1:
        mid = lo + ((hi - lo) // 2)
        yield from _oem_sort(lo, mid)
        yield from _oem_sort(mid + 1, hi)
        yield from _oem_merge(lo, hi, 1)


_SORT32 = tuple(_oem_sort(0, 31))


def _sortnet32(vals):
    vals = list(vals)
    for i, j in _SORT32:
        a, b = vals[i], vals[j]
        vals[i] = jnp.minimum(a, b)
        vals[j] = jnp.maximum(a, b)
    return vals


def _median65(vals):
    A = _sortnet32(vals[0:32])
    Bv = _sortnet32(vals[32:64])
    e = vals[64]
    u31 = jnp.minimum(Bv[31], A[31])
    for i in range(1, 32):
        u31 = jnp.minimum(u31, jnp.maximum(A[i - 1], Bv[31 - i]))
    u32 = jnp.maximum(A[0], Bv[31])
    for i in range(2, 33):
        u32 = jnp.minimum(u32, jnp.maximum(A[i - 1], Bv[32 - i]))
    return jnp.minimum(jnp.maximum(u31, e), u32)


_MM_BLK = 256


def _matmul_body(s_ref, x_ref, o_ref):
    o_ref[...] = lax.dot_general(
        s_ref[...], x_ref[...], (((0,), (1,)), ((), ())),
        preferred_element_type=jnp.float32,
    )


def _matmul_call(S0, x2):
    BF, N_ = x2.shape
    grid = (N_ // _MM_BLK,)
    return pl.pallas_call(
        _matmul_body,
        grid=grid,
        in_specs=[
            pl.BlockSpec((N_, _MM_BLK), lambda i: (0, i)),
            pl.BlockSpec((BF, N_), lambda i: (0, 0)),
        ],
        out_specs=pl.BlockSpec((_MM_BLK, BF), lambda i: (i, 0)),
        out_shape=jax.ShapeDtypeStruct((N_, BF), jnp.float32),
    )(S0, x2)


_NC, _NS = 2, 16
_NW = _NC * _NS
_CHUNK = 104


def _make_gather(rows_total, row_len):
    per_w = rows_total // _NW
    npair = per_w // _CHUNK // 2
    mesh = plsc.VectorSubcoreMesh(core_axis_name="c", subcore_axis_name="s")

    @functools.partial(
        pl.kernel,
        out_type=jax.ShapeDtypeStruct((rows_total, row_len), jnp.float32),
        mesh=mesh,
        scratch_types=[
            pltpu.VMEM((per_w,), jnp.int32),
            pltpu.VMEM((_CHUNK, row_len), jnp.float32),
            pltpu.VMEM((_CHUNK, row_len), jnp.float32),
            pltpu.SemaphoreType.DMA,
            pltpu.SemaphoreType.DMA,
            pltpu.SemaphoreType.DMA,
            pltpu.SemaphoreType.DMA,
        ],
    )
    def gather_k(table_hbm, idx_hbm, out_hbm, idx_all, rows0, rows1,
                 sg0, sg1, ss0, ss1):
        wid = lax.axis_index("s") * _NC + lax.axis_index("c")
        base = pl.multiple_of(wid * per_w, 8)
        pltpu.sync_copy(idx_hbm.at[pl.ds(base, per_w)], idx_all)

        def g_start(c, buf, sem):
            return pltpu.async_copy(
                table_hbm.at[idx_all.at[pl.ds(c * _CHUNK, _CHUNK)]], buf, sem)

        def g_wait(c, buf, sem):
            pltpu.make_async_copy(
                table_hbm.at[idx_all.at[pl.ds(c * _CHUNK, _CHUNK)]], buf, sem
            ).wait()

        def s_start(c, buf, sem):
            off = pl.multiple_of(base + c * _CHUNK, 8)
            return pltpu.async_copy(buf, out_hbm.at[pl.ds(off, _CHUNK)], sem)

        def s_wait(c, buf, sem):
            off = pl.multiple_of(base + c * _CHUNK, 8)
            pltpu.make_async_copy(buf, out_hbm.at[pl.ds(off, _CHUNK)], sem).wait()

        g_start(0, rows0, sg0)
        g_start(1, rows1, sg1)

        def body(i, carry):
            c0 = i * 2
            c1 = c0 + 1
            g_wait(c0, rows0, sg0)
            s_start(c0, rows0, ss0)
            g_wait(c1, rows1, sg1)
            s_start(c1, rows1, ss1)

            @pl.when(i < npair - 1)
            def _():
                s_wait(c0, rows0, ss0)
                g_start(c0 + 2, rows0, sg0)
                s_wait(c1, rows1, ss1)
                g_start(c1 + 2, rows1, sg1)

            return carry

        lax.fori_loop(0, npair, body, 0)
        s_wait(2 * npair - 2, rows0, ss0)
        s_wait(2 * npair - 1, rows1, ss1)

    return gather_k


_MED_SUB = 64
_MED_OUT = 128
_MED_INNER = _MED_OUT // _MED_SUB


def _median_body(w_ref, g_ref, x_ref, o_ref, acc_ref):
    j = pl.program_id(1)
    vals = [g_ref[0, d] for d in range(65)]
    med = _median65(vals)
    acc_ref[pl.ds(j * _MED_SUB, _MED_SUB), :] = med

    @pl.when(j == _MED_INNER - 1)
    def _():
        w0 = w_ref[0, 0]
        w1 = w_ref[0, 1]
        o_ref[...] = w0 * jnp.maximum(x_ref[...], 0.0) + w1 * acc_ref[...].T


def _median_call(weight, G, x2):
    NB, Kp1, _, BF = G.shape
    N_ = NB * _MED_SUB
    grid = (N_ // _MED_OUT, _MED_INNER)
    return pl.pallas_call(
        _median_body,
        grid=grid,
        in_specs=[
            pl.BlockSpec((1, 2), lambda i, j: (0, 0)),
            pl.BlockSpec((1, Kp1, _MED_SUB, BF),
                         lambda i, j: (i * _MED_INNER + j, 0, 0, 0)),
            pl.BlockSpec((BF, _MED_OUT), lambda i, j: (0, i)),
        ],
        out_specs=pl.BlockSpec((BF, _MED_OUT), lambda i, j: (0, i)),
        out_shape=jax.ShapeDtypeStruct((BF, N_), jnp.float32),
        scratch_shapes=[pltpu.VMEM((_MED_OUT, BF), jnp.float32)],
    )(weight, G, x2)


def kernel(x, S, weight, neigh_idx):
    B_, F_, N_ = x.shape
    BF = B_ * F_
    Dp1 = neigh_idx.shape[1]

    x2 = x.reshape(BF, N_)
    cur_T = _matmul_call(S[0], x2)
    idx_flat = neigh_idx.reshape(N_ // _MED_SUB, _MED_SUB, Dp1)
    idx_flat = idx_flat.transpose(0, 2, 1).reshape(-1)
    gather_k = _make_gather(Dp1 * N_, BF)
    G = gather_k(cur_T, idx_flat).reshape(N_ // _MED_SUB, Dp1, _MED_SUB, BF)
    out2 = _median_call(weight, G, x2)
    return out2.reshape(B_, F_, N_)


# speedup vs baseline: 1.8727x; 1.2145x over previous
import functools

import jax
import jax.numpy as jnp
from jax import lax
from jax.experimental import pallas as pl
from jax.experimental.pallas import tpu as pltpu
from jax.experimental.pallas import tpu_sc as plsc


def _oem_merge(lo, hi, r):
    step = r * 2
    if step < hi - lo:
        yield from _oem_merge(lo, hi, step)
        yield from _oem_merge(lo + r, hi, step)
        for i in range(lo + r, hi - r, step):
            yield (i, i + r)
    else:
        yield (lo, lo + r)


def _oem_sort(lo, hi):
    if (hi - lo) >= 1:
        mid = lo + ((hi - lo) // 2)
        yield from _oem_sort(lo, mid)
        yield from _oem_sort(mid + 1, hi)
        yield from _oem_merge(lo, hi, 1)


_SORT32 = tuple(_oem_sort(0, 31))


def _sortnet32(vals):
    vals = list(vals)
    for i, j in _SORT32:
        a, b = vals[i], vals[j]
        vals[i] = jnp.minimum(a, b)
        vals[j] = jnp.maximum(a, b)
    return vals


def _median65(vals):
    A = _sortnet32(vals[0:32])
    Bv = _sortnet32(vals[32:64])
    e = vals[64]
    u31 = jnp.minimum(Bv[31], A[31])
    for i in range(1, 32):
        u31 = jnp.minimum(u31, jnp.maximum(A[i - 1], Bv[31 - i]))
    u32 = jnp.maximum(A[0], Bv[31])
    for i in range(2, 33):
        u32 = jnp.minimum(u32, jnp.maximum(A[i - 1], Bv[32 - i]))
    return jnp.minimum(jnp.maximum(u31, e), u32)


def _bf16_bits_rtne(a):
    u = lax.bitcast_convert_type(a, jnp.int32)
    rounded = u + 0x7FFF + ((u >> 16) & 1)
    return lax.shift_right_logical(rounded, 16)


_MM_BLK = 256


def _matmul_body(s_ref, x_ref, o_ref):
    res = lax.dot_general(
        s_ref[...], x_ref[...], (((0,), (1,)), ((), ())),
        preferred_element_type=jnp.float32,
    )
    half = res.shape[1] // 2
    lo = _bf16_bits_rtne(res[:, :half])
    hi = _bf16_bits_rtne(res[:, half:])
    word = lo | lax.shift_left(hi, 16)
    o_ref[...] = lax.bitcast_convert_type(word, jnp.float32)


def _matmul_call(S0, x2):
    BF, N_ = x2.shape
    grid = (N_ // _MM_BLK,)
    return pl.pallas_call(
        _matmul_body,
        grid=grid,
        in_specs=[
            pl.BlockSpec((N_, _MM_BLK), lambda i: (0, i)),
            pl.BlockSpec((BF, N_), lambda i: (0, 0)),
        ],
        out_specs=pl.BlockSpec((_MM_BLK, BF // 2), lambda i: (i, 0)),
        out_shape=jax.ShapeDtypeStruct((N_, BF // 2), jnp.float32),
    )(S0, x2)


_NC, _NS = 2, 16
_NW = _NC * _NS
_CHUNK = 104


def _make_gather(rows_total, row_len):
    per_w = rows_total // _NW
    npair = per_w // _CHUNK // 2
    mesh = plsc.VectorSubcoreMesh(core_axis_name="c", subcore_axis_name="s")

    @functools.partial(
        pl.kernel,
        out_type=jax.ShapeDtypeStruct((rows_total, row_len), jnp.float32),
        mesh=mesh,
        scratch_types=[
            pltpu.VMEM((per_w,), jnp.int32),
            pltpu.VMEM((_CHUNK, row_len), jnp.float32),
            pltpu.VMEM((_CHUNK, row_len), jnp.float32),
            pltpu.SemaphoreType.DMA,
            pltpu.SemaphoreType.DMA,
            pltpu.SemaphoreType.DMA,
            pltpu.SemaphoreType.DMA,
        ],
    )
    def gather_k(table_hbm, idx_hbm, out_hbm, idx_all, rows0, rows1,
                 sg0, sg1, ss0, ss1):
        wid = lax.axis_index("s") * _NC + lax.axis_index("c")
        base = pl.multiple_of(wid * per_w, 8)
        pltpu.sync_copy(idx_hbm.at[pl.ds(base, per_w)], idx_all)

        def g_start(c, buf, sem):
            return pltpu.async_copy(
                table_hbm.at[idx_all.at[pl.ds(c * _CHUNK, _CHUNK)]], buf, sem)

        def g_wait(c, buf, sem):
            pltpu.make_async_copy(
                table_hbm.at[idx_all.at[pl.ds(c * _CHUNK, _CHUNK)]], buf, sem
            ).wait()

        def s_start(c, buf, sem):
            off = pl.multiple_of(base + c * _CHUNK, 8)
            return pltpu.async_copy(buf, out_hbm.at[pl.ds(off, _CHUNK)], sem)

        def s_wait(c, buf, sem):
            off = pl.multiple_of(base + c * _CHUNK, 8)
            pltpu.make_async_copy(buf, out_hbm.at[pl.ds(off, _CHUNK)], sem).wait()

        g_start(0, rows0, sg0)
        g_start(1, rows1, sg1)

        def body(i, carry):
            c0 = i * 2
            c1 = c0 + 1
            g_wait(c0, rows0, sg0)
            s_start(c0, rows0, ss0)
            g_wait(c1, rows1, sg1)
            s_start(c1, rows1, ss1)

            @pl.when(i < npair - 1)
            def _():
                s_wait(c0, rows0, ss0)
                g_start(c0 + 2, rows0, sg0)
                s_wait(c1, rows1, ss1)
                g_start(c1 + 2, rows1, sg1)

            return carry

        lax.fori_loop(0, npair, body, 0)
        s_wait(2 * npair - 2, rows0, ss0)
        s_wait(2 * npair - 1, rows1, ss1)

    return gather_k


_MED_SUB = 64
_MED_OUT = 128
_MED_INNER = _MED_OUT // _MED_SUB

_HI_MASK = -65536


def _median_body(w_ref, g_ref, x_ref, o_ref, acc_lo, acc_hi):
    j = pl.program_id(1)
    words = [lax.bitcast_convert_type(g_ref[0, d], jnp.int32) for d in range(65)]
    lo_vals = [lax.bitcast_convert_type(lax.shift_left(w, 16), jnp.float32)
               for w in words]
    hi_vals = [lax.bitcast_convert_type(w & _HI_MASK, jnp.float32)
               for w in words]
    acc_lo[pl.ds(j * _MED_SUB, _MED_SUB), :] = _median65(lo_vals)
    acc_hi[pl.ds(j * _MED_SUB, _MED_SUB), :] = _median65(hi_vals)

    @pl.when(j == _MED_INNER - 1)
    def _():
        w0 = w_ref[0, 0]
        w1 = w_ref[0, 1]
        half = x_ref.shape[0] // 2
        o_ref[:half, :] = (w0 * jnp.maximum(x_ref[:half, :], 0.0)
                           + w1 * acc_lo[...].T)
        o_ref[half:, :] = (w0 * jnp.maximum(x_ref[half:, :], 0.0)
                           + w1 * acc_hi[...].T)


def _median_call(weight, Gw, x2):
    NB, Kp1, _, BFH = Gw.shape
    BF = BFH * 2
    N_ = NB * _MED_SUB
    grid = (N_ // _MED_OUT, _MED_INNER)
    return pl.pallas_call(
        _median_body,
        grid=grid,
        in_specs=[
            pl.BlockSpec((1, 2), lambda i, j: (0, 0)),
            pl.BlockSpec((1, Kp1, _MED_SUB, BFH),
                         lambda i, j: (i * _MED_INNER + j, 0, 0, 0)),
            pl.BlockSpec((BF, _MED_OUT), lambda i, j: (0, i)),
        ],
        out_specs=pl.BlockSpec((BF, _MED_OUT), lambda i, j: (0, i)),
        out_shape=jax.ShapeDtypeStruct((BF, N_), jnp.float32),
        scratch_shapes=[
            pltpu.VMEM((_MED_OUT, BFH), jnp.float32),
            pltpu.VMEM((_MED_OUT, BFH), jnp.float32),
        ],
    )(weight, Gw, x2)


def kernel(x, S, weight, neigh_idx):
    B_, F_, N_ = x.shape
    BF = B_ * F_
    Dp1 = neigh_idx.shape[1]

    x2 = x.reshape(BF, N_)
    table = _matmul_call(S[0], x2)
    idx_flat = neigh_idx.reshape(N_ // _MED_SUB, _MED_SUB, Dp1)
    idx_flat = idx_flat.transpose(0, 2, 1).reshape(-1)
    gather_k = _make_gather(Dp1 * N_, BF // 2)
    Gw = gather_k(table, idx_flat)
    Gw = Gw.reshape(N_ // _MED_SUB, Dp1, _MED_SUB, BF // 2)
    out2 = _median_call(weight, Gw, x2)
    return out2.reshape(B_, F_, N_)
